# Initial kernel scaffold; baseline (speedup 1.0000x reference)
import jax
import jax.numpy as jnp
from jax.experimental import pallas as pl


def kernel(query, key, value, mask):
    raise NotImplementedError("write your pallas kernel here")


# speedup vs baseline: 32.7691x; 32.7691x over previous
import functools

import jax
import jax.numpy as jnp
from jax.experimental import pallas as pl
from jax.experimental.pallas import tpu as pltpu

_B, _C, _T, _H, _W = 4, 64, 10, 32, 32
_Q = _H * _W
_K = _T * _Q
_TEMP = 0.07
_TOPK = 10
_QT = 128
_NQT = _Q // _QT
_CH = 256
_NCH = _Q // _CH
_NCHUNK = _T * _NCH
_NEG = float("-inf")


def _attn_kernel(q_ref, k_ref, v0_ref, mask_ref, out_ref, s_ref, m_ref):
    qn = q_ref[0]
    mk = mask_ref[...]

    for t in range(_T):
        st = jax.lax.dot_general(
            k_ref[0, t], qn, (((0,), (0,)), ((), ())),
            precision=jax.lax.Precision.DEFAULT,
            preferred_element_type=jnp.float32)
        s_ref[t] = jnp.where(mk, st / _TEMP, _NEG)

    m_ref[...] = jnp.full((_NCHUNK, 16, _QT), _NEG, jnp.float32)

    def chunk_body(c, carry):
        t = c // _NCH
        jq = c - t * _NCH
        z = s_ref[pl.ds(t, 1), pl.ds(jq * _CH, _CH), :].reshape(_CH, _QT)
        for i in range(_TOPK):
            m = jnp.max(z, axis=0, keepdims=True)
            m_ref[pl.ds(c, 1), pl.ds(i, 1), :] = m.reshape(1, 1, _QT)
            if i < _TOPK - 1:
                z = jnp.where(z == m, _NEG, z)
        return carry

    jax.lax.fori_loop(0, _NCHUNK, chunk_body, 0)

    z = m_ref[...].reshape(_NCHUNK * 16, _QT)
    m1 = jnp.max(z, axis=0, keepdims=True)
    t10 = m1
    for _ in range(_TOPK - 1):
        z = jnp.where(z == t10, _NEG, z)
        t10 = jnp.max(z, axis=0, keepdims=True)

    nacc = jnp.zeros((1, _QT), jnp.float32)
    for t in range(_T):
        st = s_ref[t]
        w = jnp.where(st >= t10, jnp.exp(st - m1), 0.0)
        s_ref[t] = w
        nacc = nacc + jnp.sum(w, axis=0, keepdims=True)

    oacc = jnp.zeros((_C, _QT), jnp.float32)
    for t in range(_T):
        wn = s_ref[t] / nacc
        oacc = oacc + jax.lax.dot_general(
            v0_ref[t], wn, (((1,), (0,)), ((), ())),
            precision=jax.lax.Precision.DEFAULT,
            preferred_element_type=jnp.float32)
    out_ref[0] = oacc


@functools.partial(jax.jit, static_argnames=())
def kernel(query, key, value, mask):
    qn = query / jnp.maximum(jnp.linalg.norm(query, axis=1, keepdims=True), 1e-12)
    kn = key / jnp.maximum(jnp.linalg.norm(key, axis=1, keepdims=True), 1e-12)
    q = qn.reshape(_B, _C, _Q)
    k = kn.transpose(0, 2, 1, 3, 4).reshape(_B, _T, _C, _Q)
    v0 = value[0].transpose(1, 0, 2, 3).reshape(_T, _C, _Q)
    out = pl.pallas_call(
        _attn_kernel,
        grid=(_B, _NQT),
        in_specs=[
            pl.BlockSpec((1, _C, _QT), lambda b, j: (b, 0, j)),
            pl.BlockSpec((1, _T, _C, _Q), lambda b, j: (b, 0, 0, 0)),
            pl.BlockSpec((_T, _C, _Q), lambda b, j: (0, 0, 0)),
            pl.BlockSpec((_Q, _QT), lambda b, j: (0, j)),
        ],
        out_specs=pl.BlockSpec((1, _C, _QT), lambda b, j: (b, 0, j)),
        out_shape=jax.ShapeDtypeStruct((_B, _C, _Q), jnp.float32),
        scratch_shapes=[
            pltpu.VMEM((_T, _Q, _QT), jnp.float32),
            pltpu.VMEM((_NCHUNK, 16, _QT), jnp.float32),
        ],
        compiler_params=pltpu.CompilerParams(
            dimension_semantics=("arbitrary", "arbitrary"),
            vmem_limit_bytes=100 * 1024 * 1024,
        ),
    )(q, k, v0, mask)
    return out.reshape(_B, _C, _H, _W)
